# Initial kernel scaffold; baseline (speedup 1.0000x reference)
import jax
import jax.numpy as jnp
from jax.experimental import pallas as pl


def kernel(img_feas, point_feas, li_index, li_xyz, Wq, bq, Wk, bk, Wv, bv, Wo, bo):
    raise NotImplementedError("write your pallas kernel here")


# speedup vs baseline: 17.5972x; 17.5972x over previous
import functools
import math

import jax
import jax.numpy as jnp
from jax import lax
from jax.experimental import pallas as pl

B, N, K = 2, 4096, 4
PC, QKV, CI = 64, 128, 64
H = W = 128
HW = H * W
T = 512
NT = N // T
HIGHEST = lax.Precision.HIGHEST


def _proj_knn_body(img_ref, pf_ref, xq_ref, xs_ref,
                   wq_ref, bq_ref, wk_ref, bk_ref, wv_ref, bv_ref, wo_ref,
                   s_ref, u_ref, knn_ref):
    bf = jnp.bfloat16
    img = img_ref[0]
    imgb = img.astype(bf)
    pfb = pf_ref[0].astype(bf)
    qp = lax.dot_general(pfb, wq_ref[...].astype(bf),
                         (((0,), (0,)), ((), ())),
                         preferred_element_type=jnp.float32) + bq_ref[...]
    kp = jnp.dot(imgb, wk_ref[...].astype(bf),
                 preferred_element_type=jnp.float32) + bk_ref[...]
    s = jnp.sum(qp * kp, axis=1) * (1.0 / math.sqrt(QKV))
    s_ref[0, 0] = s
    v1 = jnp.dot(imgb, wv_ref[...].astype(bf),
                 preferred_element_type=jnp.float32) + bv_ref[...]
    u_ref[0] = jnp.dot(v1.astype(bf), wo_ref[...].astype(bf),
                       preferred_element_type=jnp.float32)

    q8 = xq_ref[0]
    st = xs_ref[0]
    sq_q = jnp.sum(q8 * q8, axis=1)
    sq_s = jnp.sum(st * st, axis=0)
    qs = lax.dot_general(q8.astype(bf), st.astype(bf),
                         (((1,), (0,)), ((), ())),
                         preferred_element_type=jnp.float32)
    d = sq_q[:, None] + sq_s[None, :] - 2.0 * qs
    it = lax.broadcasted_iota(jnp.int32, (T, N), 1)
    cols = []
    for _ in range(K):
        mv = jnp.min(d, axis=1, keepdims=True)
        im = jnp.where(d == mv, it, jnp.int32(2**30))
        ik = jnp.min(im, axis=1)
        cols.append(ik[:, None])
        d = jnp.where(it == ik[:, None], jnp.inf, d)
    knn_ref[0] = jnp.concatenate(cols, axis=1)


def _proj_knn(img_ff, point_feas, xyz_q, xyz_sT, Wq, bq, Wk, bk, Wv, bv, Wo):
    grid = (B, NT)
    return pl.pallas_call(
        _proj_knn_body,
        grid=grid,
        in_specs=[
            pl.BlockSpec((1, T, CI), lambda b, t: (b, t, 0)),
            pl.BlockSpec((1, PC, T), lambda b, t: (b, 0, t)),
            pl.BlockSpec((1, T, 8), lambda b, t: (b, t, 0)),
            pl.BlockSpec((1, 8, N), lambda b, t: (b, 0, 0)),
            pl.BlockSpec((PC, QKV), lambda b, t: (0, 0)),
            pl.BlockSpec((QKV,), lambda b, t: (0,)),
            pl.BlockSpec((CI, QKV), lambda b, t: (0, 0)),
            pl.BlockSpec((QKV,), lambda b, t: (0,)),
            pl.BlockSpec((CI, QKV), lambda b, t: (0, 0)),
            pl.BlockSpec((QKV,), lambda b, t: (0,)),
            pl.BlockSpec((QKV, CI), lambda b, t: (0, 0)),
        ],
        out_specs=[
            pl.BlockSpec((1, 1, T), lambda b, t: (b, 0, t)),
            pl.BlockSpec((1, T, CI), lambda b, t: (b, t, 0)),
            pl.BlockSpec((1, T, K), lambda b, t: (b, t, 0)),
        ],
        out_shape=[
            jax.ShapeDtypeStruct((B, 1, N), jnp.float32),
            jax.ShapeDtypeStruct((B, N, CI), jnp.float32),
            jax.ShapeDtypeStruct((B, N, K), jnp.int32),
        ],
    )(img_ff, point_feas, xyz_q, xyz_sT, Wq, bq, Wk, bk, Wv, bv, Wo)


def _combine_body(sg_ref, ug_ref, bo_ref, out_ref):
    sg = sg_ref[0]
    m = jnp.max(sg, axis=1, keepdims=True)
    e = jnp.exp(sg - m)
    wgt = e / jnp.sum(e, axis=1, keepdims=True)
    acc = wgt[:, 0:1] * ug_ref[0, :, 0, :]
    for k in range(1, K):
        acc = acc + wgt[:, k:k + 1] * ug_ref[0, :, k, :]
    ret = 0.25 * acc + bo_ref[...]
    out_ref[0] = ret.T


def _combine(s_g, u_g, bo):
    grid = (B, NT)
    return pl.pallas_call(
        _combine_body,
        grid=grid,
        in_specs=[
            pl.BlockSpec((1, T, K), lambda b, t: (b, t, 0)),
            pl.BlockSpec((1, T, K, CI), lambda b, t: (b, t, 0, 0)),
            pl.BlockSpec((CI,), lambda b, t: (0,)),
        ],
        out_specs=pl.BlockSpec((1, CI, T), lambda b, t: (b, 0, t)),
        out_shape=jax.ShapeDtypeStruct((B, CI, N), jnp.float32),
    )(s_g, u_g, bo)


def kernel(img_feas, point_feas, li_index, li_xyz, Wq, bq, Wk, bk, Wv, bv, Wo, bo):
    flat_img = jnp.transpose(img_feas, (0, 2, 3, 1)).reshape(B, HW, CI)
    xyz_q = jnp.pad(li_xyz, ((0, 0), (0, 0), (0, 5)))
    xyz_sT = jnp.transpose(xyz_q, (0, 2, 1))

    img_ff = jnp.take_along_axis(
        flat_img, li_index[:, :, None].astype(jnp.int32), axis=1)

    s3, u, knn = _proj_knn(img_ff, point_feas, xyz_q, xyz_sT,
                           Wq, bq, Wk, bk, Wv, bv, Wo)
    s = s3[:, 0, :]

    s_g = jnp.take_along_axis(
        jnp.broadcast_to(s[:, :, None], (B, N, K)), knn, axis=1)
    u_g = jnp.take_along_axis(
        jnp.broadcast_to(u[:, :, None, :], (B, N, K, CI)),
        jnp.broadcast_to(knn[:, :, :, None], (B, N, K, CI)), axis=1)

    return _combine(s_g, u_g, bo)


# speedup vs baseline: 29.5191x; 1.6775x over previous
import functools
import math

import jax
import jax.numpy as jnp
from jax import lax
from jax.experimental import pallas as pl
from jax.experimental.pallas import tpu as pltpu
from jax.experimental.pallas import tpu_sc as plsc

B, N, K = 2, 4096, 4
PC, QKV, CI = 64, 128, 64
H = W = 128
HW = H * W
T = 512
NT = N // T

_NC, _NS, _L = 2, 16, 16
_NW = _NC * _NS
_BP = (B * N) // _NW


@functools.cache
def _sc_kernels():
    mesh = plsc.VectorSubcoreMesh(core_axis_name="c", subcore_axis_name="s")

    @functools.partial(
        pl.kernel, mesh=mesh,
        out_type=jax.ShapeDtypeStruct((B * N, CI), jnp.float32),
        compiler_params=pltpu.CompilerParams(use_tc_tiling_on_sc=False),
        scratch_types=[
            pltpu.VMEM((_BP,), jnp.int32),
            pltpu.VMEM((_BP, CI), jnp.float32),
            pltpu.SemaphoreType.DMA,
        ],
    )
    def _img_gather(tab_hbm, idx_hbm, out_hbm, idx_v, rows_v, sem):
        wid = lax.axis_index("s") * _NC + lax.axis_index("c")
        base = wid * _BP
        pltpu.sync_copy(idx_hbm.at[pl.ds(base, _BP)], idx_v)
        boff = (base // N) * HW
        for c in range(_BP // _L):
            sl = pl.ds(c * _L, _L)
            idx_v[sl] = idx_v[sl] + jnp.broadcast_to(boff, (_L,))
        pltpu.async_copy(tab_hbm.at[idx_v], rows_v, sem).wait()
        pltpu.sync_copy(rows_v, out_hbm.at[pl.ds(base, _BP)])

    @functools.partial(
        pl.kernel, mesh=mesh,
        out_type=(
            jax.ShapeDtypeStruct((B * N * K,), jnp.float32),
            jax.ShapeDtypeStruct((B * N * K, CI), jnp.float32),
        ),
        compiler_params=pltpu.CompilerParams(use_tc_tiling_on_sc=False,
                                             needs_layout_passes=False),
        scratch_types=[
            pltpu.VMEM((_BP * K,), jnp.int32),
            pltpu.VMEM((_BP * K,), jnp.float32),
            pltpu.VMEM((_BP * K, CI), jnp.float32),
            pltpu.VMEM((B * N,), jnp.float32),
            pltpu.SemaphoreType.DMA,
        ],
    )
    def _knn_gather(knn_hbm, s_hbm, u_hbm, sg_hbm, ug_hbm,
                    idx_v, sg_v, urows_v, sv, sem):
        wid = lax.axis_index("s") * _NC + lax.axis_index("c")
        base = wid * _BP
        bi = base * K
        pltpu.sync_copy(knn_hbm.at[pl.ds(bi, _BP * K)], idx_v)
        pltpu.sync_copy(s_hbm, sv)
        boff = (base // N) * N
        for c in range((_BP * K) // _L):
            sl = pl.ds(c * _L, _L)
            v = idx_v[sl] + jnp.broadcast_to(boff, (_L,))
            idx_v[sl] = v
            sg_v[sl] = plsc.load_gather(sv, [v])
        pltpu.async_copy(u_hbm.at[idx_v], urows_v, sem).wait()
        pltpu.sync_copy(sg_v, sg_hbm.at[pl.ds(bi, _BP * K)])
        pltpu.sync_copy(urows_v, ug_hbm.at[pl.ds(bi, _BP * K)])

    return _img_gather, _knn_gather


def _proj_knn_body(img_ref, pf_ref, xq_ref, xs_ref,
                   wq_ref, bq_ref, wk_ref, bk_ref, wv_ref, bv_ref, wo_ref,
                   s_ref, u_ref, knn_ref):
    bf = jnp.bfloat16
    img = img_ref[0]
    imgb = img.astype(bf)
    pfb = pf_ref[0].astype(bf)
    qp = lax.dot_general(pfb, wq_ref[...].astype(bf),
                         (((0,), (0,)), ((), ())),
                         preferred_element_type=jnp.float32) + bq_ref[...]
    kp = jnp.dot(imgb, wk_ref[...].astype(bf),
                 preferred_element_type=jnp.float32) + bk_ref[...]
    s = jnp.sum(qp * kp, axis=1) * (1.0 / math.sqrt(QKV))
    s_ref[0, 0] = s
    v1 = jnp.dot(imgb, wv_ref[...].astype(bf),
                 preferred_element_type=jnp.float32) + bv_ref[...]
    u_ref[0] = jnp.dot(v1.astype(bf), wo_ref[...].astype(bf),
                       preferred_element_type=jnp.float32)

    q8 = xq_ref[0]
    st = xs_ref[0]
    sq_q = jnp.sum(q8 * q8, axis=1)
    sq_s = jnp.sum(st * st, axis=0)
    qs = lax.dot_general(q8.astype(bf), st.astype(bf),
                         (((1,), (0,)), ((), ())),
                         preferred_element_type=jnp.float32)
    d = sq_q[:, None] + sq_s[None, :] - 2.0 * qs
    it = lax.broadcasted_iota(jnp.int32, (T, N), 1)
    cols = []
    for _ in range(K):
        mv = jnp.min(d, axis=1, keepdims=True)
        im = jnp.where(d == mv, it, jnp.int32(2**30))
        ik = jnp.min(im, axis=1)
        cols.append(ik[:, None])
        d = jnp.where(it == ik[:, None], jnp.inf, d)
    knn_ref[0] = jnp.concatenate(cols, axis=1)


def _proj_knn(img_ff, point_feas, xyz_q, xyz_sT, Wq, bq, Wk, bk, Wv, bv, Wo):
    grid = (B, NT)
    return pl.pallas_call(
        _proj_knn_body,
        grid=grid,
        in_specs=[
            pl.BlockSpec((1, T, CI), lambda b, t: (b, t, 0)),
            pl.BlockSpec((1, PC, T), lambda b, t: (b, 0, t)),
            pl.BlockSpec((1, T, 8), lambda b, t: (b, t, 0)),
            pl.BlockSpec((1, 8, N), lambda b, t: (b, 0, 0)),
            pl.BlockSpec((PC, QKV), lambda b, t: (0, 0)),
            pl.BlockSpec((QKV,), lambda b, t: (0,)),
            pl.BlockSpec((CI, QKV), lambda b, t: (0, 0)),
            pl.BlockSpec((QKV,), lambda b, t: (0,)),
            pl.BlockSpec((CI, QKV), lambda b, t: (0, 0)),
            pl.BlockSpec((QKV,), lambda b, t: (0,)),
            pl.BlockSpec((QKV, CI), lambda b, t: (0, 0)),
        ],
        out_specs=[
            pl.BlockSpec((1, 1, T), lambda b, t: (b, 0, t)),
            pl.BlockSpec((1, T, CI), lambda b, t: (b, t, 0)),
            pl.BlockSpec((1, T, K), lambda b, t: (b, t, 0)),
        ],
        out_shape=[
            jax.ShapeDtypeStruct((B, 1, N), jnp.float32),
            jax.ShapeDtypeStruct((B, N, CI), jnp.float32),
            jax.ShapeDtypeStruct((B, N, K), jnp.int32),
        ],
    )(img_ff, point_feas, xyz_q, xyz_sT, Wq, bq, Wk, bk, Wv, bv, Wo)


def _combine_body(sg_ref, ug_ref, bo_ref, out_ref):
    sg = sg_ref[0]
    m = jnp.max(sg, axis=1, keepdims=True)
    e = jnp.exp(sg - m)
    wgt = e / jnp.sum(e, axis=1, keepdims=True)
    acc = wgt[:, 0:1] * ug_ref[0, :, 0, :]
    for k in range(1, K):
        acc = acc + wgt[:, k:k + 1] * ug_ref[0, :, k, :]
    ret = 0.25 * acc + bo_ref[...]
    out_ref[0] = ret.T


def _combine(s_g, u_g, bo):
    grid = (B, NT)
    return pl.pallas_call(
        _combine_body,
        grid=grid,
        in_specs=[
            pl.BlockSpec((1, T, K), lambda b, t: (b, t, 0)),
            pl.BlockSpec((1, T, K, CI), lambda b, t: (b, t, 0, 0)),
            pl.BlockSpec((CI,), lambda b, t: (0,)),
        ],
        out_specs=pl.BlockSpec((1, CI, T), lambda b, t: (b, 0, t)),
        out_shape=jax.ShapeDtypeStruct((B, CI, N), jnp.float32),
    )(s_g, u_g, bo)


def kernel(img_feas, point_feas, li_index, li_xyz, Wq, bq, Wk, bk, Wv, bv, Wo, bo):
    flat_img = jnp.transpose(img_feas, (0, 2, 3, 1)).reshape(B, HW, CI)
    xyz_q = jnp.pad(li_xyz, ((0, 0), (0, 0), (0, 5)))
    xyz_sT = jnp.transpose(xyz_q, (0, 2, 1))

    _img_gather, _knn_gather = _sc_kernels()

    img_ff = _img_gather(flat_img.reshape(B * HW, CI),
                         li_index.reshape(B * N)).reshape(B, N, CI)

    s3, u, knn = _proj_knn(img_ff, point_feas, xyz_q, xyz_sT,
                           Wq, bq, Wk, bk, Wv, bv, Wo)

    sg_f, ug_f = _knn_gather(knn.reshape(B * N * K), s3.reshape(B * N),
                             u.reshape(B * N, CI))
    s_g = sg_f.reshape(B, N, K)
    u_g = ug_f.reshape(B, N, K, CI)

    return _combine(s_g, u_g, bo)


# speedup vs baseline: 32.9319x; 1.1156x over previous
import functools
import math

import jax
import jax.numpy as jnp
from jax import lax
from jax.experimental import pallas as pl
from jax.experimental.pallas import tpu as pltpu
from jax.experimental.pallas import tpu_sc as plsc

B, N, K = 2, 4096, 4
PC, QKV, CI = 64, 128, 64
H = W = 128
HW = H * W
BN = B * N
T = 512
NT = N // T
_BF = jnp.bfloat16

_NC, _NS, _L = 2, 16, 16
_NW = _NC * _NS
_BP = BN // _NW


@functools.cache
def _sc_kernels():
    mesh = plsc.VectorSubcoreMesh(core_axis_name="c", subcore_axis_name="s")
    nlp = pltpu.CompilerParams(use_tc_tiling_on_sc=False,
                               needs_layout_passes=False)

    @functools.partial(
        pl.kernel, mesh=mesh,
        out_type=jax.ShapeDtypeStruct((BN, CI), jnp.float32),
        compiler_params=pltpu.CompilerParams(use_tc_tiling_on_sc=False),
        scratch_types=[
            pltpu.VMEM((_BP,), jnp.int32),
            pltpu.VMEM((_BP, CI), jnp.float32),
            pltpu.SemaphoreType.DMA,
        ],
    )
    def _img_gather(tab_hbm, idx_hbm, out_hbm, idx_v, rows_v, sem):
        wid = lax.axis_index("s") * _NC + lax.axis_index("c")
        base = wid * _BP
        pltpu.sync_copy(idx_hbm.at[pl.ds(base, _BP)], idx_v)
        boff = (base // N) * HW
        for c in range(_BP // _L):
            sl = pl.ds(c * _L, _L)
            idx_v[sl] = idx_v[sl] + jnp.broadcast_to(boff, (_L,))
        pltpu.async_copy(tab_hbm.at[idx_v], rows_v, sem).wait()
        pltpu.sync_copy(rows_v, out_hbm.at[pl.ds(base, _BP)])

    @functools.partial(
        pl.kernel, mesh=mesh,
        out_type=(
            (jax.ShapeDtypeStruct((BN * K,), jnp.float32),)
            + tuple(jax.ShapeDtypeStruct((BN, CI), jnp.float32)
                    for _ in range(K))
        ),
        compiler_params=nlp,
        scratch_types=[
            pltpu.VMEM((_BP * K,), jnp.int32),
            pltpu.VMEM((_BP * K,), jnp.int32),
            pltpu.VMEM((_BP * K,), jnp.float32),
            pltpu.VMEM((_BP * K, CI), jnp.float32),
            pltpu.VMEM((BN,), jnp.float32),
            pltpu.SemaphoreType.DMA,
        ],
    )
    def _knn_gather(knn_hbm, s_hbm, u_hbm, sg_hbm, ug0, ug1, ug2, ug3,
                    idx_v, idxk_v, sg_v, urows_v, sv, sem):
        ugs = (ug0, ug1, ug2, ug3)
        wid = lax.axis_index("s") * _NC + lax.axis_index("c")
        base = wid * _BP
        bi = base * K
        pltpu.sync_copy(knn_hbm.at[pl.ds(bi, _BP * K)], idx_v)
        pltpu.sync_copy(s_hbm, sv)
        boff = (base // N) * N
        for c in range((_BP * K) // _L):
            sl = pl.ds(c * _L, _L)
            v = idx_v[sl] + jnp.broadcast_to(boff, (_L,))
            idx_v[sl] = v
            sg_v[sl] = plsc.load_gather(sv, [v])
        lane = lax.iota(jnp.int32, _L)
        for k in range(K):
            for c in range(_BP // _L):
                src = jnp.broadcast_to(c * _L * K + k, (_L,)) + lane * K
                idxk_v[pl.ds(k * _BP + c * _L, _L)] = (
                    plsc.load_gather(idx_v, [src]))
        pltpu.async_copy(u_hbm.at[idxk_v], urows_v, sem).wait()
        pltpu.sync_copy(sg_v, sg_hbm.at[pl.ds(bi, _BP * K)])
        for k in range(K):
            pltpu.sync_copy(urows_v.at[pl.ds(k * _BP, _BP)],
                            ugs[k].at[pl.ds(base, _BP)])

    return _img_gather, _knn_gather


def _knn_body(xq_ref, xs_ref, knn_ref):
    q8 = xq_ref[0]
    st = xs_ref[0]
    sq_q = jnp.sum(q8 * q8, axis=1)
    sq_s = jnp.sum(st * st, axis=0)
    qs = lax.dot_general(q8.astype(_BF), st.astype(_BF),
                         (((1,), (0,)), ((), ())),
                         preferred_element_type=jnp.float32)
    d = sq_q[:, None] + sq_s[None, :] - 2.0 * qs
    itf = lax.broadcasted_iota(jnp.int32, (T, N), 1).astype(jnp.float32)
    cols = []
    for k in range(K):
        ik = jnp.argmin(d, axis=1).astype(jnp.float32)[:, None]
        cols.append(ik)
        if k < K - 1:
            d = jnp.where(itf == ik, jnp.inf, d)
    knn_ref[...] = jnp.concatenate(cols, axis=1).astype(jnp.int32)


def _knn_call(xyz_q, xyz_sT):
    return pl.pallas_call(
        _knn_body,
        grid=(B, NT),
        in_specs=[
            pl.BlockSpec((1, T, 8), lambda b, t: (b, t, 0)),
            pl.BlockSpec((1, 8, N), lambda b, t: (b, 0, 0)),
        ],
        out_specs=pl.BlockSpec((T, K), lambda b, t: (b * NT + t, 0)),
        out_shape=jax.ShapeDtypeStruct((BN, K), jnp.int32),
    )(xyz_q, xyz_sT)


def _proj_body(img_ref, pf_ref, wq_ref, bq_ref, wk_ref, bk_ref,
               wv_ref, bv_ref, wo_ref, s_ref, u_ref):
    img = img_ref[...]
    imgb = img.astype(_BF)
    pfb = pf_ref[0].astype(_BF)
    qp = lax.dot_general(pfb, wq_ref[...].astype(_BF),
                         (((0,), (0,)), ((), ())),
                         preferred_element_type=jnp.float32) + bq_ref[...]
    kp = jnp.dot(imgb, wk_ref[...].astype(_BF),
                 preferred_element_type=jnp.float32) + bk_ref[...]
    s_ref[...] = jnp.sum(qp * kp, axis=1) * (1.0 / math.sqrt(QKV))
    v1 = jnp.dot(imgb, wv_ref[...].astype(_BF),
                 preferred_element_type=jnp.float32) + bv_ref[...]
    u_ref[...] = jnp.dot(v1.astype(_BF), wo_ref[...].astype(_BF),
                         preferred_element_type=jnp.float32)


def _proj_call(img_ff, point_feas, Wq, bq, Wk, bk, Wv, bv, Wo):
    return pl.pallas_call(
        _proj_body,
        grid=(B, NT),
        in_specs=[
            pl.BlockSpec((T, CI), lambda b, t: (b * NT + t, 0)),
            pl.BlockSpec((1, PC, T), lambda b, t: (b, 0, t)),
            pl.BlockSpec((PC, QKV), lambda b, t: (0, 0)),
            pl.BlockSpec((QKV,), lambda b, t: (0,)),
            pl.BlockSpec((CI, QKV), lambda b, t: (0, 0)),
            pl.BlockSpec((QKV,), lambda b, t: (0,)),
            pl.BlockSpec((CI, QKV), lambda b, t: (0, 0)),
            pl.BlockSpec((QKV,), lambda b, t: (0,)),
            pl.BlockSpec((QKV, CI), lambda b, t: (0, 0)),
        ],
        out_specs=[
            pl.BlockSpec((T,), lambda b, t: (b * NT + t,)),
            pl.BlockSpec((T, CI), lambda b, t: (b * NT + t, 0)),
        ],
        out_shape=[
            jax.ShapeDtypeStruct((BN,), jnp.float32),
            jax.ShapeDtypeStruct((BN, CI), jnp.float32),
        ],
    )(img_ff, point_feas, Wq, bq, Wk, bk, Wv, bv, Wo)


def _combine_body(sg_ref, ug0_ref, ug1_ref, ug2_ref, ug3_ref, bo_ref,
                  out_ref):
    sg = sg_ref[...]
    m = jnp.max(sg, axis=1, keepdims=True)
    e = jnp.exp(sg - m)
    wgt = e / jnp.sum(e, axis=1, keepdims=True)
    ug = (ug0_ref, ug1_ref, ug2_ref, ug3_ref)
    acc = wgt[:, 0:1] * ug[0][...]
    for k in range(1, K):
        acc = acc + wgt[:, k:k + 1] * ug[k][...]
    ret = 0.25 * acc + bo_ref[...]
    out_ref[0] = ret.T


def _combine(sg2, ugs, bo):
    return pl.pallas_call(
        _combine_body,
        grid=(B, NT),
        in_specs=[pl.BlockSpec((T, K), lambda b, t: (b * NT + t, 0))]
        + [pl.BlockSpec((T, CI), lambda b, t: (b * NT + t, 0))
           for _ in range(K)]
        + [pl.BlockSpec((CI,), lambda b, t: (0,))],
        out_specs=pl.BlockSpec((1, CI, T), lambda b, t: (b, 0, t)),
        out_shape=jax.ShapeDtypeStruct((B, CI, N), jnp.float32),
    )(sg2, *ugs, bo)


def kernel(img_feas, point_feas, li_index, li_xyz, Wq, bq, Wk, bk, Wv, bv, Wo, bo):
    flat_img = jnp.transpose(img_feas, (0, 2, 3, 1)).reshape(B * HW, CI)
    xyz_q = jnp.pad(li_xyz, ((0, 0), (0, 0), (0, 5)))
    xyz_sT = jnp.transpose(xyz_q, (0, 2, 1))

    _img_gather, _knn_gather = _sc_kernels()

    knn2 = _knn_call(xyz_q, xyz_sT)

    img_ff = _img_gather(flat_img, li_index.reshape(BN))

    s, u = _proj_call(img_ff, point_feas, Wq, bq, Wk, bk, Wv, bv, Wo)

    sg_f, *ugs = _knn_gather(knn2.reshape(BN * K), s, u)
    sg2 = sg_f.reshape(BN, K)

    return _combine(sg2, ugs, bo)


# speedup vs baseline: 36.7677x; 1.1165x over previous
import functools
import math

import jax
import jax.numpy as jnp
from jax import lax
from jax.experimental import pallas as pl
from jax.experimental.pallas import tpu as pltpu
from jax.experimental.pallas import tpu_sc as plsc

B, N, K = 2, 4096, 4
PC, QKV, CI = 64, 128, 64
H = W = 128
HW = H * W
BN = B * N
T = 512
NT = N // T
_BF = jnp.bfloat16

_NC, _NS, _L = 2, 16, 16
_NW = _NC * _NS
_BP = BN // _NW


@functools.cache
def _sc_kernels():
    mesh = plsc.VectorSubcoreMesh(core_axis_name="c", subcore_axis_name="s")
    nlp = pltpu.CompilerParams(use_tc_tiling_on_sc=False,
                               needs_layout_passes=False)

    @functools.partial(
        pl.kernel, mesh=mesh,
        out_type=jax.ShapeDtypeStruct((BN, CI), jnp.float32),
        compiler_params=pltpu.CompilerParams(use_tc_tiling_on_sc=False),
        scratch_types=[
            pltpu.VMEM((_BP,), jnp.int32),
            pltpu.VMEM((_BP, CI), jnp.float32),
            pltpu.SemaphoreType.DMA,
        ],
    )
    def _img_gather(tab_hbm, idx_hbm, out_hbm, idx_v, rows_v, sem):
        wid = lax.axis_index("s") * _NC + lax.axis_index("c")
        base = wid * _BP
        pltpu.sync_copy(idx_hbm.at[pl.ds(base, _BP)], idx_v)
        boff = (base // N) * HW
        for c in range(_BP // _L):
            sl = pl.ds(c * _L, _L)
            idx_v[sl] = idx_v[sl] + jnp.broadcast_to(boff, (_L,))
        pltpu.async_copy(tab_hbm.at[idx_v], rows_v, sem).wait()
        pltpu.sync_copy(rows_v, out_hbm.at[pl.ds(base, _BP)])

    @functools.partial(
        pl.kernel, mesh=mesh,
        out_type=jax.ShapeDtypeStruct((BN, CI), jnp.float32),
        compiler_params=nlp,
        scratch_types=[
            pltpu.VMEM((_BP * K,), jnp.int32),
            pltpu.VMEM((_BP * K,), jnp.int32),
            pltpu.VMEM((_BP * K,), jnp.float32),
            pltpu.VMEM((_BP * K,), jnp.float32),
            pltpu.VMEM((_BP * K, CI), jnp.float32),
            pltpu.VMEM((BN,), jnp.float32),
            pltpu.VMEM((CI,), jnp.float32),
            pltpu.VMEM((_BP, CI), jnp.float32),
            pltpu.SemaphoreType.DMA,
        ],
    )
    def _knn_attend(knn_hbm, s_hbm, u_hbm, bo_hbm, ret_hbm,
                    idx_v, idxk_v, sg_v, w_v, urows_v, sv, bo_v, ret_v, sem):
        wid = lax.axis_index("s") * _NC + lax.axis_index("c")
        base = wid * _BP
        bi = base * K
        pltpu.sync_copy(knn_hbm.at[pl.ds(bi, _BP * K)], idx_v)
        pltpu.sync_copy(s_hbm, sv)
        pltpu.sync_copy(bo_hbm, bo_v)
        boff = (base // N) * N
        for c in range((_BP * K) // _L):
            sl = pl.ds(c * _L, _L)
            idx_v[sl] = idx_v[sl] + jnp.broadcast_to(boff, (_L,))
        lane = lax.iota(jnp.int32, _L)
        for k in range(K):
            for c in range(_BP // _L):
                src = jnp.broadcast_to(c * _L * K + k, (_L,)) + lane * K
                v = plsc.load_gather(idx_v, [src])
                idxk_v[pl.ds(k * _BP + c * _L, _L)] = v
                sg_v[pl.ds(k * _BP + c * _L, _L)] = (
                    plsc.load_gather(sv, [v]))
        pltpu.async_copy(u_hbm.at[idxk_v], urows_v, sem).wait()
        for c in range(_BP // _L):
            sl = [pl.ds(k * _BP + c * _L, _L) for k in range(K)]
            e = [sg_v[sl[k]] for k in range(K)]
            m = jnp.maximum(jnp.maximum(e[0], e[1]),
                            jnp.maximum(e[2], e[3]))
            e = [jnp.exp(x - m) for x in e]
            z = ((e[0] + e[1]) + e[2]) + e[3]
            for k in range(K):
                w_v[sl[k]] = e[k] / z
        def body(p, _):
            acc = [jnp.zeros((_L,), jnp.float32) for _ in range(CI // _L)]
            for k in range(K):
                wspl = plsc.load_gather(
                    w_v, [jnp.broadcast_to(k * _BP + p, (_L,))])
                row = k * _BP + p
                for ch in range(CI // _L):
                    acc[ch] = acc[ch] + wspl * urows_v[row,
                                                       pl.ds(ch * _L, _L)]
            for ch in range(CI // _L):
                csl = pl.ds(ch * _L, _L)
                ret_v[p, csl] = 0.25 * acc[ch] + bo_v[csl]
            return 0
        lax.fori_loop(0, _BP, body, 0)
        pltpu.sync_copy(ret_v, ret_hbm.at[pl.ds(base, _BP)])

    return _img_gather, _knn_attend


def _knn_body(xq_ref, xs_ref, knn_ref):
    q8 = xq_ref[0]
    st = xs_ref[0]
    sq_q = jnp.sum(q8 * q8, axis=1)
    sq_s = jnp.sum(st * st, axis=0)
    qs = lax.dot_general(q8.astype(_BF), st.astype(_BF),
                         (((1,), (0,)), ((), ())),
                         preferred_element_type=jnp.float32)
    d = sq_q[:, None] + sq_s[None, :] - 2.0 * qs
    itf = lax.broadcasted_iota(jnp.int32, (T, N), 1).astype(jnp.float32)
    cols = []
    for k in range(K):
        ik = jnp.argmin(d, axis=1).astype(jnp.float32)[:, None]
        cols.append(ik)
        if k < K - 1:
            d = jnp.where(itf == ik, jnp.inf, d)
    knn_ref[...] = jnp.concatenate(cols, axis=1).astype(jnp.int32)


def _knn_call(xyz_q, xyz_sT):
    return pl.pallas_call(
        _knn_body,
        grid=(B, NT),
        in_specs=[
            pl.BlockSpec((1, T, 8), lambda b, t: (b, t, 0)),
            pl.BlockSpec((1, 8, N), lambda b, t: (b, 0, 0)),
        ],
        out_specs=pl.BlockSpec((T, K), lambda b, t: (b * NT + t, 0)),
        out_shape=jax.ShapeDtypeStruct((BN, K), jnp.int32),
    )(xyz_q, xyz_sT)


def _proj_body(img_ref, pf_ref, wq_ref, bq_ref, wk_ref, bk_ref,
               wv_ref, bv_ref, wo_ref, s_ref, u_ref):
    img = img_ref[...]
    imgb = img.astype(_BF)
    pfb = pf_ref[0].astype(_BF)
    qp = lax.dot_general(pfb, wq_ref[...].astype(_BF),
                         (((0,), (0,)), ((), ())),
                         preferred_element_type=jnp.float32) + bq_ref[...]
    kp = jnp.dot(imgb, wk_ref[...].astype(_BF),
                 preferred_element_type=jnp.float32) + bk_ref[...]
    s_ref[...] = jnp.sum(qp * kp, axis=1) * (1.0 / math.sqrt(QKV))
    v1 = jnp.dot(imgb, wv_ref[...].astype(_BF),
                 preferred_element_type=jnp.float32) + bv_ref[...]
    u_ref[...] = jnp.dot(v1.astype(_BF), wo_ref[...].astype(_BF),
                         preferred_element_type=jnp.float32)


def _proj_call(img_ff, point_feas, Wq, bq, Wk, bk, Wv, bv, Wo):
    return pl.pallas_call(
        _proj_body,
        grid=(B, NT),
        in_specs=[
            pl.BlockSpec((T, CI), lambda b, t: (b * NT + t, 0)),
            pl.BlockSpec((1, PC, T), lambda b, t: (b, 0, t)),
            pl.BlockSpec((PC, QKV), lambda b, t: (0, 0)),
            pl.BlockSpec((QKV,), lambda b, t: (0,)),
            pl.BlockSpec((CI, QKV), lambda b, t: (0, 0)),
            pl.BlockSpec((QKV,), lambda b, t: (0,)),
            pl.BlockSpec((CI, QKV), lambda b, t: (0, 0)),
            pl.BlockSpec((QKV,), lambda b, t: (0,)),
            pl.BlockSpec((QKV, CI), lambda b, t: (0, 0)),
        ],
        out_specs=[
            pl.BlockSpec((T,), lambda b, t: (b * NT + t,)),
            pl.BlockSpec((T, CI), lambda b, t: (b * NT + t, 0)),
        ],
        out_shape=[
            jax.ShapeDtypeStruct((BN,), jnp.float32),
            jax.ShapeDtypeStruct((BN, CI), jnp.float32),
        ],
    )(img_ff, point_feas, Wq, bq, Wk, bk, Wv, bv, Wo)


def kernel(img_feas, point_feas, li_index, li_xyz, Wq, bq, Wk, bk, Wv, bv, Wo, bo):
    flat_img = jnp.transpose(img_feas, (0, 2, 3, 1)).reshape(B * HW, CI)
    xyz_q = jnp.pad(li_xyz, ((0, 0), (0, 0), (0, 5)))
    xyz_sT = jnp.transpose(xyz_q, (0, 2, 1))

    _img_gather, _knn_attend = _sc_kernels()

    knn2 = _knn_call(xyz_q, xyz_sT)

    img_ff = _img_gather(flat_img, li_index.reshape(BN))

    s, u = _proj_call(img_ff, point_feas, Wq, bq, Wk, bk, Wv, bv, Wo)

    ret = _knn_attend(knn2.reshape(BN * K), s, u, bo)

    return jnp.transpose(ret.reshape(B, N, CI), (0, 2, 1))
